# baseline (speedup 1.0000x reference)
import jax
import jax.numpy as jnp
from jax import lax
from jax.experimental import pallas as pl
from jax.experimental.pallas import tpu as pltpu
from jax.experimental.pallas import tpu_sc as plsc

_NUM_CLASSES = 1000
_HIDDEN = 1024
_BATCH = 16384

_NC = 2
_NS = 16
_NW = _NC * _NS
_LANES = 16

_B_PER_W = _BATCH // _NW
_CHUNK = 16
_NCHUNK = _B_PER_W // _CHUNK

_ROWS = _NUM_CLASSES + 1
_STAGE = 64
_VPR = _HIDDEN // _LANES


def _embed_body(labels_hbm, drop_hbm, table_hbm, out_hbm,
                table_s, drop_v, idx_v, row0, row1, tb0, tb1,
                g0, g1, w0, w1, ssem):
    rowbufs = (row0, row1)
    tilebufs = (tb0, tb1)
    gsems = (g0, g1)
    wsems = (w0, w1)

    sid = lax.axis_index("s")
    wid = sid * _NC + lax.axis_index("c")
    base = wid * _B_PER_W

    @pl.when(sid < 15)
    def _():
        off = sid * (_STAGE * _HIDDEN)
        pltpu.async_copy(table_hbm.at[pl.ds(off, _STAGE * _HIDDEN)],
                         table_s.at[pl.ds(off, _STAGE * _HIDDEN)], ssem)

    @pl.when(sid == 15)
    def _():
        tail = (_ROWS - 15 * _STAGE) * _HIDDEN
        off = 15 * _STAGE * _HIDDEN
        pltpu.async_copy(table_hbm.at[pl.ds(off, tail)],
                         table_s.at[pl.ds(off, tail)], ssem)

    pltpu.sync_copy(labels_hbm.at[pl.ds(base, _B_PER_W)], idx_v)
    pltpu.sync_copy(drop_hbm.at[pl.ds(base, _B_PER_W)], drop_v)

    for i in range(_B_PER_W // _LANES):
        sl = pl.ds(i * _LANES, _LANES)
        idx_v[sl] = jnp.where(drop_v[sl] == 1, jnp.int32(_NUM_CLASSES),
                              idx_v[sl])

    @pl.when(sid < 15)
    def _():
        pltpu.make_async_copy(
            table_hbm.at[pl.ds(0, _STAGE * _HIDDEN)],
            table_s.at[pl.ds(0, _STAGE * _HIDDEN)], ssem).wait()

    @pl.when(sid == 15)
    def _():
        tail = (_ROWS - 15 * _STAGE) * _HIDDEN
        pltpu.make_async_copy(table_hbm.at[pl.ds(0, tail)],
                              table_s.at[pl.ds(0, tail)], ssem).wait()

    plsc.subcore_barrier()

    def start_gather(chunk, slot):
        vec = idx_v[pl.ds(chunk * _CHUNK, _LANES)]
        for j in range(_LANES):
            row = vec[j]
            off = pl.multiple_of(row * _HIDDEN, _HIDDEN)
            pltpu.async_copy(table_s.at[pl.ds(off, _HIDDEN)],
                             rowbufs[slot].at[pl.ds(j * _HIDDEN, _HIDDEN)],
                             gsems[slot])

    def wait_gather(slot):
        pltpu.make_async_copy(
            table_hbm.at[pl.ds(0, _CHUNK * _HIDDEN)], rowbufs[slot],
            gsems[slot]).wait()

    def permute(slot):
        for j in range(_CHUNK):
            for w in range(_VPR):
                v = rowbufs[slot][pl.ds(j * _HIDDEN + w * _LANES, _LANES)]
                tilebufs[slot][j, pl.ds(w * _LANES, _LANES)] = v

    def start_write(chunk, slot):
        b0 = pl.multiple_of(base + chunk * _CHUNK, _CHUNK)
        pltpu.async_copy(tilebufs[slot], out_hbm.at[pl.ds(b0, _CHUNK)],
                         wsems[slot])

    def wait_write(slot):
        pltpu.make_async_copy(tilebufs[slot], out_hbm.at[pl.ds(0, _CHUNK)],
                              wsems[slot]).wait()

    start_gather(0, 0)
    wait_gather(0)

    def step(g, _):
        a = 2 * g
        b = 2 * g + 1

        @pl.when(g >= 1)
        def _():
            wait_write(0)

        permute(0)
        start_write(a, 0)


        @pl.when(g >= 1)
        def _():
            wait_write(1)

        permute(1)
        start_write(b, 1)
        return _

    lax.fori_loop(0, _NCHUNK // 2, step, None)
    wait_write(0)
    wait_write(1)


@jax.jit
def kernel(labels, force_drop_ids, embedding_table):
    labels = labels.astype(jnp.int32)
    drops = force_drop_ids.astype(jnp.int32)
    table_flat = embedding_table.reshape(-1)
    mesh = plsc.VectorSubcoreMesh(core_axis_name="c", subcore_axis_name="s")
    run = pl.kernel(
        _embed_body,
        out_type=jax.ShapeDtypeStruct((_BATCH, _HIDDEN), jnp.float32),
        mesh=mesh,
        scratch_types=[
            pltpu.VMEM_SHARED((_ROWS * _HIDDEN,), jnp.float32),
            pltpu.VMEM((_B_PER_W,), jnp.int32),
            pltpu.VMEM((_B_PER_W,), jnp.int32),
            pltpu.VMEM((_CHUNK * _HIDDEN,), jnp.float32),
            pltpu.VMEM((_CHUNK * _HIDDEN,), jnp.float32),
            pltpu.VMEM((_CHUNK, _HIDDEN), jnp.float32),
            pltpu.VMEM((_CHUNK, _HIDDEN), jnp.float32),
            pltpu.SemaphoreType.DMA,
            pltpu.SemaphoreType.DMA,
            pltpu.SemaphoreType.DMA,
            pltpu.SemaphoreType.DMA,
            pltpu.SemaphoreType.DMA,
        ],
    )
    return run(labels, drops, table_flat)


# speedup vs baseline: 1.4032x; 1.4032x over previous
import jax
import jax.numpy as jnp
from jax import lax
from jax.experimental import pallas as pl
from jax.experimental.pallas import tpu as pltpu
from jax.experimental.pallas import tpu_sc as plsc

_NUM_CLASSES = 1000
_HIDDEN = 1024
_BATCH = 16384

_NC = 2
_NS = 16
_NW = _NC * _NS
_LANES = 16

_B_PER_W = _BATCH // _NW
_CHUNK = 16
_NCHUNK = _B_PER_W // _CHUNK

_ROWS = _NUM_CLASSES + 1
_STAGE = 64
_PIECES = _HIDDEN // 128


def _embed_body(labels_hbm, drop_hbm, table_hbm, out_hbm,
                table_s, drop_v, idx_v, row0, row1,
                g0, g1, w0, w1, ssem):
    rowbufs = (row0, row1)
    gsems = (g0, g1)
    wsems = (w0, w1)

    sid = lax.axis_index("s")
    wid = sid * _NC + lax.axis_index("c")
    base = wid * _B_PER_W

    @pl.when(sid < 15)
    def _():
        off = sid * (_STAGE * _PIECES)
        pltpu.async_copy(table_hbm.at[pl.ds(off, _STAGE * _PIECES)],
                         table_s.at[pl.ds(off, _STAGE * _PIECES)], ssem)

    @pl.when(sid == 15)
    def _():
        tail = (_ROWS - 15 * _STAGE) * _PIECES
        off = 15 * _STAGE * _PIECES
        pltpu.async_copy(table_hbm.at[pl.ds(off, tail)],
                         table_s.at[pl.ds(off, tail)], ssem)

    pltpu.sync_copy(labels_hbm.at[pl.ds(base, _B_PER_W)], idx_v)
    pltpu.sync_copy(drop_hbm.at[pl.ds(base, _B_PER_W)], drop_v)

    for i in range(_B_PER_W // _LANES):
        sl = pl.ds(i * _LANES, _LANES)
        idx_v[sl] = jnp.where(drop_v[sl] == 1, jnp.int32(_NUM_CLASSES),
                              idx_v[sl])

    @pl.when(sid < 15)
    def _():
        pltpu.make_async_copy(
            table_hbm.at[pl.ds(0, _STAGE * _PIECES)],
            table_s.at[pl.ds(0, _STAGE * _PIECES)], ssem).wait()

    @pl.when(sid == 15)
    def _():
        tail = (_ROWS - 15 * _STAGE) * _PIECES
        pltpu.make_async_copy(table_hbm.at[pl.ds(0, tail)],
                              table_s.at[pl.ds(0, tail)], ssem).wait()

    plsc.subcore_barrier()

    def start_gather(chunk, slot):
        vec = idx_v[pl.ds(chunk * _CHUNK, _LANES)]
        for j in range(_LANES):
            row = vec[j]
            off = pl.multiple_of(row * _PIECES, _PIECES)
            pltpu.async_copy(table_s.at[pl.ds(off, _PIECES)],
                             rowbufs[slot].at[j], gsems[slot])

    def wait_gather(slot):
        for j in range(_LANES):
            pltpu.make_async_copy(table_hbm.at[pl.ds(0, _PIECES)],
                                  rowbufs[slot].at[j], gsems[slot]).wait()

    def start_write(chunk, slot):
        r0 = pl.multiple_of((base + chunk * _CHUNK) // 8, 2)
        for hg in range(_CHUNK // 8):
            for c in range(_PIECES):
                pltpu.async_copy(
                    rowbufs[slot].at[pl.ds(hg * 8, 8), c],
                    out_hbm.at[r0 + hg, c], wsems[slot])

    def wait_write(slot):
        for hg in range(_CHUNK // 8):
            for c in range(_PIECES):
                pltpu.make_async_copy(
                    rowbufs[slot].at[pl.ds(hg * 8, 8), c],
                    out_hbm.at[0, c], wsems[slot]).wait()

    start_gather(0, 0)

    def step(g, _):
        a = 2 * g
        b = 2 * g + 1

        wait_gather(0)
        start_gather(b, 1)

        @pl.when(g >= 1)
        def _():
            wait_write(0)

        start_write(a, 0)

        wait_gather(1)

        @pl.when(g < _NCHUNK // 2 - 1)
        def _():
            start_gather(a + 2, 0)

        @pl.when(g >= 1)
        def _():
            wait_write(1)

        start_write(b, 1)
        return _

    lax.fori_loop(0, _NCHUNK // 2, step, None)
    wait_write(0)
    wait_write(1)


@jax.jit
def kernel(labels, force_drop_ids, embedding_table):
    labels = labels.astype(jnp.int32)
    drops = force_drop_ids.astype(jnp.int32)
    table_v = embedding_table.reshape(_ROWS * 8, 128)
    mesh = plsc.VectorSubcoreMesh(core_axis_name="c", subcore_axis_name="s")
    run = pl.kernel(
        _embed_body,
        out_type=jax.ShapeDtypeStruct((_BATCH // 8, _PIECES, 8, 128),
                                      jnp.float32),
        mesh=mesh,
        scratch_types=[
            pltpu.VMEM_SHARED((_ROWS * 8, 128), jnp.float32),
            pltpu.VMEM((_B_PER_W,), jnp.int32),
            pltpu.VMEM((_B_PER_W,), jnp.int32),
            pltpu.VMEM((_CHUNK, 8, 128), jnp.float32),
            pltpu.VMEM((_CHUNK, 8, 128), jnp.float32),
            pltpu.SemaphoreType.DMA,
            pltpu.SemaphoreType.DMA,
            pltpu.SemaphoreType.DMA,
            pltpu.SemaphoreType.DMA,
            pltpu.SemaphoreType.DMA,
        ],
    )
    out4 = run(labels, drops, table_v)
    return out4.transpose(0, 2, 1, 3).reshape(_BATCH, _HIDDEN)


# speedup vs baseline: 1.4076x; 1.0032x over previous
import jax
import jax.numpy as jnp
from jax import lax
from jax.experimental import pallas as pl
from jax.experimental.pallas import tpu as pltpu
from jax.experimental.pallas import tpu_sc as plsc

_NUM_CLASSES = 1000
_HIDDEN = 1024
_BATCH = 16384

_NC = 2
_NS = 16
_NW = _NC * _NS
_LANES = 16

_B_PER_W = _BATCH // _NW
_CHUNK = 16
_NCHUNK = _B_PER_W // _CHUNK

_ROWS = _NUM_CLASSES + 1
_STAGE = 64
_PIECES = _HIDDEN // 128


def _embed_body(labels_hbm, drop_hbm, table_hbm, out_hbm,
                table_s, drop_v, idx_v, row0, row1,
                g0, g1, w0, w1, ssem):
    rowbufs = (row0, row1)
    gsems = (g0, g1)
    wsems = (w0, w1)

    sid = lax.axis_index("s")
    wid = sid * _NC + lax.axis_index("c")
    base = wid * _B_PER_W

    @pl.when(sid < 15)
    def _():
        off = sid * (_STAGE * _PIECES)
        pltpu.async_copy(table_hbm.at[pl.ds(off, _STAGE * _PIECES)],
                         table_s.at[pl.ds(off, _STAGE * _PIECES)], ssem)

    @pl.when(sid == 15)
    def _():
        tail = (_ROWS - 15 * _STAGE) * _PIECES
        off = 15 * _STAGE * _PIECES
        pltpu.async_copy(table_hbm.at[pl.ds(off, tail)],
                         table_s.at[pl.ds(off, tail)], ssem)

    pltpu.sync_copy(labels_hbm.at[pl.ds(base, _B_PER_W)], idx_v)
    pltpu.sync_copy(drop_hbm.at[pl.ds(base, _B_PER_W)], drop_v)

    for i in range(_B_PER_W // _LANES):
        sl = pl.ds(i * _LANES, _LANES)
        idx_v[sl] = jnp.where(drop_v[sl] == 1, jnp.int32(_NUM_CLASSES),
                              idx_v[sl])

    @pl.when(sid < 15)
    def _():
        pltpu.make_async_copy(
            table_hbm.at[pl.ds(0, _STAGE * _PIECES)],
            table_s.at[pl.ds(0, _STAGE * _PIECES)], ssem).wait()

    @pl.when(sid == 15)
    def _():
        tail = (_ROWS - 15 * _STAGE) * _PIECES
        pltpu.make_async_copy(table_hbm.at[pl.ds(0, tail)],
                              table_s.at[pl.ds(0, tail)], ssem).wait()

    plsc.subcore_barrier()

    def start_gather(chunk, slot):
        vec = idx_v[pl.ds(chunk * _CHUNK, _LANES)]
        for j in range(_LANES):
            row = vec[j]
            off = pl.multiple_of(row * _PIECES, _PIECES)
            pltpu.async_copy(table_s.at[pl.ds(off, _PIECES)],
                             rowbufs[slot].at[j // 8, j % 8], gsems[slot])

    def wait_gather(slot):
        for j in range(_LANES):
            pltpu.make_async_copy(
                table_hbm.at[pl.ds(0, _PIECES)],
                rowbufs[slot].at[j // 8, j % 8], gsems[slot]).wait()

    def start_write(chunk, slot):
        r0 = pl.multiple_of((base + chunk * _CHUNK) // 8, 2)
        for c in range(_PIECES):
            pltpu.async_copy(
                rowbufs[slot].at[:, :, c, :],
                out_hbm.at[pl.ds(r0, 2), c], wsems[slot])

    def wait_write(slot):
        for c in range(_PIECES):
            pltpu.make_async_copy(
                rowbufs[slot].at[:, :, c, :],
                out_hbm.at[pl.ds(0, 2), c], wsems[slot]).wait()

    start_gather(0, 0)

    def step(g, _):
        a = 2 * g
        b = 2 * g + 1

        wait_gather(0)
        start_gather(b, 1)

        @pl.when(g >= 1)
        def _():
            wait_write(0)

        start_write(a, 0)

        wait_gather(1)

        @pl.when(g < _NCHUNK // 2 - 1)
        def _():
            start_gather(a + 2, 0)

        @pl.when(g >= 1)
        def _():
            wait_write(1)

        start_write(b, 1)
        return _

    lax.fori_loop(0, _NCHUNK // 2, step, None)
    wait_write(0)
    wait_write(1)


@jax.jit
def kernel(labels, force_drop_ids, embedding_table):
    labels = labels.astype(jnp.int32)
    drops = force_drop_ids.astype(jnp.int32)
    table_v = embedding_table.reshape(_ROWS * 8, 128)
    mesh = plsc.VectorSubcoreMesh(core_axis_name="c", subcore_axis_name="s")
    run = pl.kernel(
        _embed_body,
        out_type=jax.ShapeDtypeStruct((_BATCH // 8, _PIECES, 8, 128),
                                      jnp.float32),
        mesh=mesh,
        scratch_types=[
            pltpu.VMEM_SHARED((_ROWS * 8, 128), jnp.float32),
            pltpu.VMEM((_B_PER_W,), jnp.int32),
            pltpu.VMEM((_B_PER_W,), jnp.int32),
            pltpu.VMEM((_CHUNK // 8, 8, 8, 128), jnp.float32),
            pltpu.VMEM((_CHUNK // 8, 8, 8, 128), jnp.float32),
            pltpu.SemaphoreType.DMA,
            pltpu.SemaphoreType.DMA,
            pltpu.SemaphoreType.DMA,
            pltpu.SemaphoreType.DMA,
            pltpu.SemaphoreType.DMA,
        ],
    )
    out4 = run(labels, drops, table_v)
    return out4.transpose(0, 2, 1, 3).reshape(_BATCH, _HIDDEN)
